# Initial kernel scaffold; baseline (speedup 1.0000x reference)
import jax
import jax.numpy as jnp
from jax.experimental import pallas as pl


def kernel(selected_mask, edge_labels, edge_batch, answer_hit):
    raise NotImplementedError("write your pallas kernel here")


# speedup vs baseline: 67.5890x; 67.5890x over previous
import functools
import math

import jax
import jax.numpy as jnp
from jax import lax
from jax.experimental import pallas as pl
from jax.experimental.pallas import tpu as pltpu
from jax.experimental.pallas import tpu_sc as plsc

LANES = 16
NUM_CORES = 2
NUM_SUBCORES = 16
NUM_WORKERS = NUM_CORES * NUM_SUBCORES

SUCCESS_REWARD = 1.0
FAILURE_REWARD = 0.01
SHAPING_COEF = 0.5
LOG_SUCCESS = math.log(SUCCESS_REWARD)
LOG_FAILURE = math.log(FAILURE_REWARD)


def _pick_block(chunk: int) -> int:
    for cand in (8192, 8000, 6400, 5120, 4096, 4000, 3200, 2048, 1600,
                 1024, 800, 512, 400, 256, 128, 64, 32, 16):
        if chunk % cand == 0:
            return cand
    raise ValueError(f"no block size divides chunk={chunk}")


def _sc_segment_body(chunk, block, num_graphs,
                     sel_hbm, lab_hbm, seg_hbm,
                     tp_out, ps_out, ts_out,
                     sel_buf, lab_buf, seg_buf,
                     acc_tp, acc_ps, acc_ts):
    wid = lax.axis_index("s") * NUM_CORES + lax.axis_index("c")
    lane = lax.iota(jnp.int32, LANES)
    l0 = lane == 0
    l15 = lane == LANES - 1
    zeros = jnp.zeros((LANES,), jnp.float32)

    def zero_body(i, carry):
        acc_tp[pl.ds(i * LANES, LANES)] = zeros
        acc_ps[pl.ds(i * LANES, LANES)] = zeros
        acc_ts[pl.ds(i * LANES, LANES)] = zeros
        return carry

    lax.fori_loop(0, num_graphs // LANES, zero_body, 0)

    def vec_body(j, carry):
        o = j * LANES
        s = seg_buf[pl.ds(o + 8, LANES)]
        sprev = seg_buf[pl.ds(o + 7, LANES)]
        snext = seg_buf[pl.ds(o + 9, LANES)]
        m_end = (s != snext) | l15
        m_start = (s != sprev) | l0
        pred = sel_buf[pl.ds(o, LANES)]
        lab = lab_buf[pl.ds(o, LANES)]
        targ = jnp.where(lab > 0.5, 1.0, 0.0).astype(jnp.float32)
        pt = pred * targ
        for acc, v in ((acc_tp, pt), (acc_ps, pred), (acc_ts, targ)):
            c = plsc.cumsum(v)
            plsc.addupdate_scatter(acc, [s], c, mask=m_end)
            plsc.addupdate_scatter(acc, [s], v - c, mask=m_start)
        return carry

    def blk_body(b, carry):
        base = wid * chunk + b * block
        pltpu.sync_copy(sel_hbm.at[pl.ds(base, block)], sel_buf)
        pltpu.sync_copy(lab_hbm.at[pl.ds(base, block)], lab_buf)
        pltpu.sync_copy(seg_hbm.at[pl.ds(base, block)],
                        seg_buf.at[pl.ds(8, block)])
        lax.fori_loop(0, block // LANES, vec_body, 0)
        return carry

    lax.fori_loop(0, chunk // block, blk_body, 0)

    pltpu.sync_copy(acc_tp, tp_out.at[wid])
    pltpu.sync_copy(acc_ps, ps_out.at[wid])
    pltpu.sync_copy(acc_ts, ts_out.at[wid])


def _segment_partials(selected_mask, edge_labels, edge_batch, num_graphs):
    num_edges = selected_mask.shape[0]
    assert num_edges % (NUM_WORKERS * LANES) == 0
    chunk = num_edges // NUM_WORKERS
    block = _pick_block(chunk)
    mesh = plsc.VectorSubcoreMesh(core_axis_name="c", subcore_axis_name="s",
                                  num_cores=NUM_CORES,
                                  num_subcores=NUM_SUBCORES)
    acc = jax.ShapeDtypeStruct((NUM_WORKERS, num_graphs), jnp.float32)
    run = pl.kernel(
        functools.partial(_sc_segment_body, chunk, block, num_graphs),
        out_type=(acc, acc, acc),
        mesh=mesh,
        compiler_params=pltpu.CompilerParams(needs_layout_passes=False),
        scratch_types=(
            pltpu.VMEM((block,), jnp.float32),
            pltpu.VMEM((block,), jnp.float32),
            pltpu.VMEM((block + 16,), jnp.int32),
            pltpu.VMEM((num_graphs,), jnp.float32),
            pltpu.VMEM((num_graphs,), jnp.float32),
            pltpu.VMEM((num_graphs,), jnp.float32),
        ),
    )
    return run(selected_mask, edge_labels, edge_batch)


def _finalize_body(tp_ref, ps_ref, ts_ref, hit_ref,
                   reward_ref, logr_ref, succ_ref,
                   prec_ref, rec_ref, f1_ref):
    tp = jnp.sum(tp_ref[...], axis=0, keepdims=True)
    ps = jnp.sum(ps_ref[...], axis=0, keepdims=True)
    ts = jnp.sum(ts_ref[...], axis=0, keepdims=True)
    zeros = jnp.zeros_like(tp)
    prec = jnp.where(ps > 0, tp / jnp.maximum(ps, 1.0), zeros)
    rec = jnp.where(ts > 0, tp / jnp.maximum(ts, 1.0), zeros)
    f1 = 2.0 * prec * rec / (prec + rec + 1e-08)
    hit = hit_ref[...]
    logr = jnp.where(hit.astype(jnp.bool_),
                     jnp.float32(LOG_SUCCESS),
                     jnp.float32(LOG_FAILURE)) + SHAPING_COEF * f1
    reward_ref[...] = jnp.exp(logr)
    logr_ref[...] = logr
    succ_ref[...] = hit.astype(jnp.float32)
    prec_ref[...] = prec
    rec_ref[...] = rec
    f1_ref[...] = f1


def _finalize(tp_p, ps_p, ts_p, hit2d):
    g = hit2d.shape[1]
    out = jax.ShapeDtypeStruct((1, g), jnp.float32)
    return pl.pallas_call(
        _finalize_body,
        out_shape=(out,) * 6,
    )(tp_p, ps_p, ts_p, hit2d)


def kernel(selected_mask, edge_labels, edge_batch, answer_hit):
    num_graphs = answer_hit.shape[0]
    tp_p, ps_p, ts_p = _segment_partials(
        selected_mask.astype(jnp.float32),
        edge_labels.astype(jnp.float32),
        edge_batch.astype(jnp.int32),
        num_graphs,
    )
    outs = _finalize(tp_p, ps_p, ts_p,
                     answer_hit.astype(jnp.int32).reshape(1, num_graphs))
    return tuple(o.reshape(num_graphs) for o in outs)


# speedup vs baseline: 83.3332x; 1.2329x over previous
import functools
import math

import jax
import jax.numpy as jnp
from jax import lax
from jax.experimental import pallas as pl
from jax.experimental.pallas import tpu as pltpu
from jax.experimental.pallas import tpu_sc as plsc

LANES = 16
NUM_CORES = 2
NUM_SUBCORES = 16
NUM_WORKERS = NUM_CORES * NUM_SUBCORES

SUCCESS_REWARD = 1.0
FAILURE_REWARD = 0.01
SHAPING_COEF = 0.5
LOG_SUCCESS = math.log(SUCCESS_REWARD)
LOG_FAILURE = math.log(FAILURE_REWARD)


def _pick_block(chunk: int) -> int:
    for cand in (8192, 8000, 6400, 5120, 4096, 4000, 3200, 2048, 1600,
                 1024, 800, 512, 400, 256, 128, 64, 32, 16):
        if chunk % cand == 0:
            return cand
    raise ValueError(f"no block size divides chunk={chunk}")


GROUP = 64


def _sc_segment_body(chunk, block, num_graphs,
                     sel_hbm, lab_hbm, seg_hbm,
                     tp_out, ps_out, ts_out,
                     sel_buf, lab_buf, seg_buf,
                     acc_tp, acc_ps, acc_ts, accv, cur_ref):
    wid = lax.axis_index("s") * NUM_CORES + lax.axis_index("c")
    lane = lax.iota(jnp.int32, LANES)
    l0 = lane == 0
    l15 = lane == LANES - 1
    zeros = jnp.zeros((LANES,), jnp.float32)

    def zero_body(i, carry):
        acc_tp[pl.ds(i * LANES, LANES)] = zeros
        acc_ps[pl.ds(i * LANES, LANES)] = zeros
        acc_ts[pl.ds(i * LANES, LANES)] = zeros
        return carry

    lax.fori_loop(0, num_graphs // LANES, zero_body, 0)
    for a in range(3):
        accv[pl.ds(a * LANES, LANES)] = zeros
    cur_ref[0] = 0

    def load_vals(o):
        pred = sel_buf[pl.ds(o, LANES)]
        lab = lab_buf[pl.ds(o, LANES)]
        tm = lab > 0.5
        targ = jnp.where(tm, 1.0, 0.0).astype(jnp.float32)
        pt = jnp.where(tm, pred, 0.0).astype(jnp.float32)
        return pt, pred, targ

    def general_vec(o):
        s = seg_buf[pl.ds(o + 8, LANES)]
        sprev = seg_buf[pl.ds(o + 7, LANES)]
        snext = seg_buf[pl.ds(o + 9, LANES)]
        m_end = (s != snext) | l15
        m_start = (s != sprev) | l0
        vals = load_vals(o)
        for acc, v in zip((acc_tp, acc_ps, acc_ts), vals):
            c = plsc.cumsum(v)
            plsc.addupdate_scatter(acc, [s], c, mask=m_end)
            plsc.addupdate_scatter(acc, [s], v - c, mask=m_start)

    def flush():
        idx = jnp.full((LANES,), cur_ref[0], dtype=jnp.int32)
        for a, acc in enumerate((acc_tp, acc_ps, acc_ts)):
            tot = jnp.sum(accv[pl.ds(a * LANES, LANES)])
            totv = jnp.full((LANES,), tot, dtype=jnp.float32)
            plsc.addupdate_scatter(acc, [idx], totv, mask=l0)
            accv[pl.ds(a * LANES, LANES)] = zeros

    def group_body(g, carry):
        o = g * GROUP
        s_first = seg_buf[pl.ds(o + 8, LANES)][0]
        s_last = seg_buf[pl.ds(o + 8 + GROUP - LANES, LANES)][LANES - 1]
        uniform = (s_first == cur_ref[0]) & (s_last == s_first)

        @pl.when(uniform)
        def _():
            for k in range(GROUP // LANES):
                pt, pred, targ = load_vals(o + k * LANES)
                plsc.addupdate(accv.at[pl.ds(0, LANES)], pt)
                plsc.addupdate(accv.at[pl.ds(LANES, LANES)], pred)
                plsc.addupdate(accv.at[pl.ds(2 * LANES, LANES)], targ)

        @pl.when(jnp.logical_not(uniform))
        def _():
            flush()
            for k in range(GROUP // LANES):
                general_vec(o + k * LANES)
            cur_ref[0] = s_last

        return carry

    def blk_body(b, carry):
        base = wid * chunk + b * block
        pltpu.sync_copy(sel_hbm.at[pl.ds(base, block)], sel_buf)
        pltpu.sync_copy(lab_hbm.at[pl.ds(base, block)], lab_buf)
        pltpu.sync_copy(seg_hbm.at[pl.ds(base, block)],
                        seg_buf.at[pl.ds(8, block)])
        lax.fori_loop(0, block // GROUP, group_body, 0)
        return carry

    lax.fori_loop(0, chunk // block, blk_body, 0)
    flush()

    pltpu.sync_copy(acc_tp, tp_out.at[wid])
    pltpu.sync_copy(acc_ps, ps_out.at[wid])
    pltpu.sync_copy(acc_ts, ts_out.at[wid])


def _segment_partials(selected_mask, edge_labels, edge_batch, num_graphs):
    num_edges = selected_mask.shape[0]
    assert num_edges % (NUM_WORKERS * LANES) == 0
    chunk = num_edges // NUM_WORKERS
    block = _pick_block(chunk)
    assert block % GROUP == 0
    mesh = plsc.VectorSubcoreMesh(core_axis_name="c", subcore_axis_name="s",
                                  num_cores=NUM_CORES,
                                  num_subcores=NUM_SUBCORES)
    acc = jax.ShapeDtypeStruct((NUM_WORKERS, num_graphs), jnp.float32)
    run = pl.kernel(
        functools.partial(_sc_segment_body, chunk, block, num_graphs),
        out_type=(acc, acc, acc),
        mesh=mesh,
        compiler_params=pltpu.CompilerParams(needs_layout_passes=False),
        scratch_types=(
            pltpu.VMEM((block,), jnp.float32),
            pltpu.VMEM((block,), jnp.float32),
            pltpu.VMEM((block + 16,), jnp.int32),
            pltpu.VMEM((num_graphs,), jnp.float32),
            pltpu.VMEM((num_graphs,), jnp.float32),
            pltpu.VMEM((num_graphs,), jnp.float32),
            pltpu.VMEM((3 * LANES,), jnp.float32),
            pltpu.SMEM((1,), jnp.int32),
        ),
    )
    return run(selected_mask, edge_labels, edge_batch)


def _finalize_body(tp_ref, ps_ref, ts_ref, hit_ref,
                   reward_ref, logr_ref, succ_ref,
                   prec_ref, rec_ref, f1_ref):
    tp = jnp.sum(tp_ref[...], axis=0, keepdims=True)
    ps = jnp.sum(ps_ref[...], axis=0, keepdims=True)
    ts = jnp.sum(ts_ref[...], axis=0, keepdims=True)
    zeros = jnp.zeros_like(tp)
    prec = jnp.where(ps > 0, tp / jnp.maximum(ps, 1.0), zeros)
    rec = jnp.where(ts > 0, tp / jnp.maximum(ts, 1.0), zeros)
    f1 = 2.0 * prec * rec / (prec + rec + 1e-08)
    hit = hit_ref[...]
    logr = jnp.where(hit.astype(jnp.bool_),
                     jnp.float32(LOG_SUCCESS),
                     jnp.float32(LOG_FAILURE)) + SHAPING_COEF * f1
    reward_ref[...] = jnp.exp(logr)
    logr_ref[...] = logr
    succ_ref[...] = hit.astype(jnp.float32)
    prec_ref[...] = prec
    rec_ref[...] = rec
    f1_ref[...] = f1


def _finalize(tp_p, ps_p, ts_p, hit2d):
    g = hit2d.shape[1]
    out = jax.ShapeDtypeStruct((1, g), jnp.float32)
    return pl.pallas_call(
        _finalize_body,
        out_shape=(out,) * 6,
    )(tp_p, ps_p, ts_p, hit2d)


def kernel(selected_mask, edge_labels, edge_batch, answer_hit):
    num_graphs = answer_hit.shape[0]
    tp_p, ps_p, ts_p = _segment_partials(
        selected_mask.astype(jnp.float32),
        edge_labels.astype(jnp.float32),
        edge_batch.astype(jnp.int32),
        num_graphs,
    )
    outs = _finalize(tp_p, ps_p, ts_p,
                     answer_hit.astype(jnp.int32).reshape(1, num_graphs))
    return tuple(o.reshape(num_graphs) for o in outs)


# speedup vs baseline: 111.3664x; 1.3364x over previous
import functools
import math

import jax
import jax.numpy as jnp
from jax import lax
from jax.experimental import pallas as pl
from jax.experimental.pallas import tpu as pltpu
from jax.experimental.pallas import tpu_sc as plsc

LANES = 16
NUM_CORES = 2
NUM_SUBCORES = 16
NUM_WORKERS = NUM_CORES * NUM_SUBCORES

SUCCESS_REWARD = 1.0
FAILURE_REWARD = 0.01
SHAPING_COEF = 0.5
LOG_SUCCESS = math.log(SUCCESS_REWARD)
LOG_FAILURE = math.log(FAILURE_REWARD)


def _pick_block(chunk: int) -> int:
    for cand in (8192, 8000, 6400, 5120, 4096, 4000, 3200, 2048, 1600,
                 1024, 800, 512, 400, 256, 128, 64, 32, 16):
        if chunk % cand == 0:
            return cand
    raise ValueError(f"no block size divides chunk={chunk}")


GROUP = 64


def _sc_segment_body(chunk, block, num_graphs,
                     sel_hbm, lab_hbm, seg_hbm,
                     tp_out, ps_out, ts_out,
                     sel_buf, lab_buf, seg_buf,
                     acc_tp, acc_ps, acc_ts, cur_ref):
    wid = lax.axis_index("s") * NUM_CORES + lax.axis_index("c")
    lane = lax.iota(jnp.int32, LANES)
    l0 = lane == 0
    l15 = lane == LANES - 1
    zeros = jnp.zeros((LANES,), jnp.float32)

    def zero_body(i, carry):
        acc_tp[pl.ds(i * LANES, LANES)] = zeros
        acc_ps[pl.ds(i * LANES, LANES)] = zeros
        acc_ts[pl.ds(i * LANES, LANES)] = zeros
        return carry

    lax.fori_loop(0, num_graphs // LANES, zero_body, 0)
    cur_ref[0] = 0

    def load_vals(o):
        pred = sel_buf[pl.ds(o, LANES)]
        lab = lab_buf[pl.ds(o, LANES)]
        tm = lab > 0.5
        targ = jnp.where(tm, 1.0, 0.0).astype(jnp.float32)
        pt = jnp.where(tm, pred, 0.0).astype(jnp.float32)
        return pt, pred, targ

    def general_vec(o):
        s = seg_buf[pl.ds(o + 8, LANES)]
        sprev = seg_buf[pl.ds(o + 7, LANES)]
        snext = seg_buf[pl.ds(o + 9, LANES)]
        m_end = (s != snext) | l15
        m_start = (s != sprev) | l0
        vals = load_vals(o)
        for acc, v in zip((acc_tp, acc_ps, acc_ts), vals):
            c = plsc.cumsum(v)
            plsc.addupdate_scatter(acc, [s], c, mask=m_end)
            plsc.addupdate_scatter(acc, [s], v - c, mask=m_start)

    def flush(a3):
        idx = jnp.full((LANES,), cur_ref[0], dtype=jnp.int32)
        for acc, av in zip((acc_tp, acc_ps, acc_ts), a3):
            tot = jnp.sum(av)
            totv = jnp.full((LANES,), tot, dtype=jnp.float32)
            plsc.addupdate_scatter(acc, [idx], totv, mask=l0)

    def group_body(g, a3):
        o = g * GROUP
        s_first = seg_buf[pl.ds(o + 8, LANES)][0]
        s_last = seg_buf[pl.ds(o + 8 + GROUP - LANES, LANES)][LANES - 1]
        uniform = (s_first == cur_ref[0]) & (s_last == s_first)

        vals = [load_vals(o + k * LANES) for k in range(GROUP // LANES)]
        sums = [(vals[0][a] + vals[1][a]) + (vals[2][a] + vals[3][a])
                for a in range(3)]

        @pl.when(jnp.logical_not(uniform))
        def _():
            flush(a3)
            for k in range(GROUP // LANES):
                general_vec(o + k * LANES)
            cur_ref[0] = s_last

        return tuple(
            jnp.where(uniform, av + sv, 0.0)
            for av, sv in zip(a3, sums))

    def blk_body(b, a3):
        base = wid * chunk + b * block
        pltpu.sync_copy(sel_hbm.at[pl.ds(base, block)], sel_buf)
        pltpu.sync_copy(lab_hbm.at[pl.ds(base, block)], lab_buf)
        pltpu.sync_copy(seg_hbm.at[pl.ds(base, block)],
                        seg_buf.at[pl.ds(8, block)])
        return lax.fori_loop(0, block // GROUP, group_body, a3)

    a3 = lax.fori_loop(0, chunk // block, blk_body, (zeros, zeros, zeros))
    flush(a3)

    pltpu.sync_copy(acc_tp, tp_out.at[wid])
    pltpu.sync_copy(acc_ps, ps_out.at[wid])
    pltpu.sync_copy(acc_ts, ts_out.at[wid])


def _segment_partials(selected_mask, edge_labels, edge_batch, num_graphs):
    num_edges = selected_mask.shape[0]
    assert num_edges % (NUM_WORKERS * LANES) == 0
    chunk = num_edges // NUM_WORKERS
    block = _pick_block(chunk)
    assert block % GROUP == 0
    mesh = plsc.VectorSubcoreMesh(core_axis_name="c", subcore_axis_name="s",
                                  num_cores=NUM_CORES,
                                  num_subcores=NUM_SUBCORES)
    acc = jax.ShapeDtypeStruct((NUM_WORKERS, num_graphs), jnp.float32)
    run = pl.kernel(
        functools.partial(_sc_segment_body, chunk, block, num_graphs),
        out_type=(acc, acc, acc),
        mesh=mesh,
        compiler_params=pltpu.CompilerParams(needs_layout_passes=False),
        scratch_types=(
            pltpu.VMEM((block,), jnp.float32),
            pltpu.VMEM((block,), jnp.float32),
            pltpu.VMEM((block + 16,), jnp.int32),
            pltpu.VMEM((num_graphs,), jnp.float32),
            pltpu.VMEM((num_graphs,), jnp.float32),
            pltpu.VMEM((num_graphs,), jnp.float32),
            pltpu.SMEM((1,), jnp.int32),
        ),
    )
    return run(selected_mask, edge_labels, edge_batch)


def _finalize_body(tp_ref, ps_ref, ts_ref, hit_ref,
                   reward_ref, logr_ref, succ_ref,
                   prec_ref, rec_ref, f1_ref):
    tp = jnp.sum(tp_ref[...], axis=0, keepdims=True)
    ps = jnp.sum(ps_ref[...], axis=0, keepdims=True)
    ts = jnp.sum(ts_ref[...], axis=0, keepdims=True)
    zeros = jnp.zeros_like(tp)
    prec = jnp.where(ps > 0, tp / jnp.maximum(ps, 1.0), zeros)
    rec = jnp.where(ts > 0, tp / jnp.maximum(ts, 1.0), zeros)
    f1 = 2.0 * prec * rec / (prec + rec + 1e-08)
    hit = hit_ref[...]
    logr = jnp.where(hit.astype(jnp.bool_),
                     jnp.float32(LOG_SUCCESS),
                     jnp.float32(LOG_FAILURE)) + SHAPING_COEF * f1
    reward_ref[...] = jnp.exp(logr)
    logr_ref[...] = logr
    succ_ref[...] = hit.astype(jnp.float32)
    prec_ref[...] = prec
    rec_ref[...] = rec
    f1_ref[...] = f1


def _finalize(tp_p, ps_p, ts_p, hit2d):
    g = hit2d.shape[1]
    out = jax.ShapeDtypeStruct((1, g), jnp.float32)
    return pl.pallas_call(
        _finalize_body,
        out_shape=(out,) * 6,
    )(tp_p, ps_p, ts_p, hit2d)


def kernel(selected_mask, edge_labels, edge_batch, answer_hit):
    num_graphs = answer_hit.shape[0]
    tp_p, ps_p, ts_p = _segment_partials(
        selected_mask.astype(jnp.float32),
        edge_labels.astype(jnp.float32),
        edge_batch.astype(jnp.int32),
        num_graphs,
    )
    outs = _finalize(tp_p, ps_p, ts_p,
                     answer_hit.astype(jnp.int32).reshape(1, num_graphs))
    return tuple(o.reshape(num_graphs) for o in outs)
